# baseline (speedup 1.0000x reference)
import functools

import jax
import jax.numpy as jnp
from jax import lax
from jax.experimental import pallas as pl
from jax.experimental.pallas import tpu as pltpu
from jax.experimental.pallas import tpu_sc as plsc

B = 16384
D = 64
NC = 2
NS = 16
NW = NC * NS
BPW = B // NW
CHUNK = 128
NCHUNK = BPW // CHUNK
TCBLK = 8192

_GATHER_DNUMS = lax.GatherDimensionNumbers(
    offset_dims=(), collapsed_slice_dims=(0,), start_index_map=(0,))


def _shuffle(x, idx):
    return lax.gather(x, idx[:, None], _GATHER_DNUMS, slice_sizes=(1,),
                      mode=lax.GatherScatterMode.PROMISE_IN_BOUNDS)


def _tc_pack_body(uT_ref, pT_ref, out_ref):
    i = lax.broadcasted_iota(jnp.int32, (D, D), 0)
    j = lax.broadcasted_iota(jnp.int32, (D, D), 1)
    eye = jnp.where(i == j, 1.0, 0.0).astype(jnp.float32)
    ut = lax.dot_general(uT_ref[...], eye, (((0,), (0,)), ((), ())),
                         precision=lax.Precision.DEFAULT)
    pt = lax.dot_general(pT_ref[...], eye, (((0,), (0,)), ((), ())),
                         precision=lax.Precision.DEFAULT)
    fused = jnp.concatenate([ut, pt], axis=1).astype(jnp.bfloat16)
    x3 = fused.reshape(TCBLK // 2, 2, 128)
    even = lax.convert_element_type(
        lax.bitcast_convert_type(x3[:, 0, :], jnp.uint16), jnp.int32)
    odd = lax.convert_element_type(
        lax.bitcast_convert_type(x3[:, 1, :], jnp.uint16), jnp.int32)
    out_ref[...] = even | lax.shift_left(odd, 16)


_tc_pack = pl.pallas_call(
    _tc_pack_body,
    grid=(123,),
    in_specs=[pl.BlockSpec((D, TCBLK), lambda j: (0, j)),
              pl.BlockSpec((D, TCBLK), lambda j: (0, j))],
    out_specs=pl.BlockSpec((TCBLK // 2, 128), lambda j: (j, 0)),
    out_shape=jax.ShapeDtypeStruct((500000, 128), jnp.int32),
)


def _sc_body(uidx_hbm, pidx_hbm, tab_hbm, out_hbm,
             uidx_v, pidx_v, urow_v, prow_v, ubuf_v, pbuf_v, out_v, sems):
    wid = lax.axis_index("s") * NC + lax.axis_index("c")
    base = wid * BPW
    lane = lax.iota(jnp.int32, 16)

    pltpu.sync_copy(uidx_hbm.at[pl.ds(wid * NCHUNK, NCHUNK)], uidx_v)
    pltpu.sync_copy(pidx_hbm.at[pl.ds(wid * NCHUNK, NCHUNK)], pidx_v)

    for c in range(NCHUNK):
        for g in range(CHUNK // 16):
            sl = pl.ds(g * 16, 16)
            urow_v[c, sl] = lax.shift_right_logical(uidx_v[c, sl], 1)
            prow_v[c, sl] = lax.shift_right_logical(pidx_v[c, sl], 1)

    def fire(c):
        ring = c % 2
        return (pltpu.async_copy(tab_hbm.at[urow_v.at[c]],
                                 ubuf_v.at[ring], sems.at[ring]),
                pltpu.async_copy(tab_hbm.at[prow_v.at[c]],
                                 pbuf_v.at[ring], sems.at[ring]))

    def compute(c):
        ring = c % 2
        ub = ubuf_v.at[ring]
        pb = pbuf_v.at[ring]

        himask = jnp.int32(-65536)

        def g_body(g, carry):
            sl = pl.ds(g * 16, 16)
            upar = uidx_v[c, sl] & 1
            ppar = pidx_v[c, sl] & 1
            out_vec = jnp.zeros((16,), jnp.float32)
            for r in range(16):
                row = g * 16 + r
                rvec = lane * 0 + r
                usel = _shuffle(upar, rvec) == 1
                psel = _shuffle(ppar, rvec) == 1
                acc = jnp.zeros((16,), jnp.float32)
                for dd in range(D // 16):
                    wu = ub[row, pl.ds(dd * 16, 16)]
                    wp = pb[row, pl.ds(D + dd * 16, 16)]
                    uval = jnp.where(
                        usel,
                        lax.bitcast_convert_type(wu & himask, jnp.float32),
                        lax.bitcast_convert_type(lax.shift_left(wu, 16),
                                                 jnp.float32))
                    pval = jnp.where(
                        psel,
                        lax.bitcast_convert_type(wp & himask, jnp.float32),
                        lax.bitcast_convert_type(lax.shift_left(wp, 16),
                                                 jnp.float32))
                    acc = acc + uval * pval
                for sh in (8, 4, 2, 1):
                    acc = acc + _shuffle(acc, lane ^ sh)
                out_vec = jnp.where(lane == r, acc, out_vec)
            out_v[pl.ds(c * CHUNK + g * 16, 16)] = out_vec
            return carry

        lax.fori_loop(0, CHUNK // 16, g_body, 0)

    inflight = [fire(0), fire(1)]
    for c in range(NCHUNK):
        for cp in inflight.pop(0):
            cp.wait()
        compute(c)
        if c + 2 < NCHUNK:
            inflight.append(fire(c + 2))

    pltpu.sync_copy(out_v, out_hbm.at[pl.ds(base, BPW)])


@jax.jit
def _collab_dot(uidx, pidx, table):
    run = functools.partial(
        pl.kernel,
        mesh=plsc.VectorSubcoreMesh(core_axis_name="c", subcore_axis_name="s"),
        compiler_params=pltpu.CompilerParams(needs_layout_passes=False),
        out_type=jax.ShapeDtypeStruct((B,), jnp.float32),
        scratch_types=[
            pltpu.VMEM((NCHUNK, CHUNK), jnp.int32),
            pltpu.VMEM((NCHUNK, CHUNK), jnp.int32),
            pltpu.VMEM((NCHUNK, CHUNK), jnp.int32),
            pltpu.VMEM((NCHUNK, CHUNK), jnp.int32),
            pltpu.VMEM((2, CHUNK, 128), jnp.int32),
            pltpu.VMEM((2, CHUNK, 128), jnp.int32),
            pltpu.VMEM((BPW,), jnp.float32),
            pltpu.SemaphoreType.DMA((2,)),
        ],
    )(_sc_body)
    return run(uidx, pidx, table)


def kernel(inputs, users_w, products_w):
    uidx = inputs[:, 0].reshape(B // CHUNK, CHUNK)
    pidx = inputs[:, 1].reshape(B // CHUNK, CHUNK)
    table = _tc_pack(users_w.T, products_w.T)
    out = _collab_dot(uidx, pidx, table)
    return out[:, None]


# speedup vs baseline: 2.0681x; 2.0681x over previous
import functools

import jax
import jax.numpy as jnp
from jax import lax
from jax.experimental import pallas as pl
from jax.experimental.pallas import tpu as pltpu
from jax.experimental.pallas import tpu_sc as plsc

B = 16384
D = 64
NC = 2
NS = 16
NW = NC * NS
BPW = B // NW
CHUNK = 128
NCHUNK = BPW // CHUNK
TCBLK = 16384

_GATHER_DNUMS = lax.GatherDimensionNumbers(
    offset_dims=(), collapsed_slice_dims=(0,), start_index_map=(0,))


def _shuffle(x, idx):
    return lax.gather(x, idx[:, None], _GATHER_DNUMS, slice_sizes=(1,),
                      mode=lax.GatherScatterMode.PROMISE_IN_BOUNDS)


def _tc_pack_body(uT_ref, pT_ref, out_ref):
    i = lax.broadcasted_iota(jnp.int32, (D, D), 0)
    j = lax.broadcasted_iota(jnp.int32, (D, D), 1)
    eye = jnp.where(i == j, 1.0, 0.0).astype(jnp.float32)
    ut = lax.dot_general(uT_ref[...], eye, (((0,), (0,)), ((), ())),
                         precision=lax.Precision.DEFAULT)
    pt = lax.dot_general(pT_ref[...], eye, (((0,), (0,)), ((), ())),
                         precision=lax.Precision.DEFAULT)
    out_ref[...] = jnp.concatenate([ut, pt], axis=1)


_tc_pack = pl.pallas_call(
    _tc_pack_body,
    grid=(62,),
    in_specs=[pl.BlockSpec((D, TCBLK), lambda j: (0, j)),
              pl.BlockSpec((D, TCBLK), lambda j: (0, j))],
    out_specs=pl.BlockSpec((TCBLK, 128), lambda j: (j, 0)),
    out_shape=jax.ShapeDtypeStruct((1000000, 128), jnp.float32),
)


def _sc_body(uidx_hbm, pidx_hbm, tab_hbm, out_hbm,
             uidx_v, pidx_v, ubuf_v, pbuf_v, out_v, sems):
    wid = lax.axis_index("s") * NC + lax.axis_index("c")
    base = wid * BPW
    lane = lax.iota(jnp.int32, 16)

    pltpu.sync_copy(uidx_hbm.at[pl.ds(wid * NCHUNK, NCHUNK)], uidx_v)
    pltpu.sync_copy(pidx_hbm.at[pl.ds(wid * NCHUNK, NCHUNK)], pidx_v)

    def fire(c):
        ring = c % 2
        return (pltpu.async_copy(tab_hbm.at[uidx_v.at[c]],
                                 ubuf_v.at[ring], sems.at[ring]),
                pltpu.async_copy(tab_hbm.at[pidx_v.at[c]],
                                 pbuf_v.at[ring], sems.at[ring]))

    def compute(c):
        ring = c % 2
        ub = ubuf_v.at[ring]
        pb = pbuf_v.at[ring]

        def g_body(g, carry):
            out_vec = jnp.zeros((16,), jnp.float32)
            for r in range(16):
                row = g * 16 + r
                acc = ub[row, pl.ds(0, 16)] * pb[row, pl.ds(D, 16)]
                for dd in range(1, D // 16):
                    acc = acc + (ub[row, pl.ds(dd * 16, 16)]
                                 * pb[row, pl.ds(D + dd * 16, 16)])
                for sh in (8, 4, 2, 1):
                    acc = acc + _shuffle(acc, lane ^ sh)
                out_vec = jnp.where(lane == r, acc, out_vec)
            out_v[pl.ds(c * CHUNK + g * 16, 16)] = out_vec
            return carry

        lax.fori_loop(0, CHUNK // 16, g_body, 0)

    inflight = [fire(0), fire(1)]
    for c in range(NCHUNK):
        for cp in inflight.pop(0):
            cp.wait()
        compute(c)
        if c + 2 < NCHUNK:
            inflight.append(fire(c + 2))

    pltpu.sync_copy(out_v, out_hbm.at[pl.ds(base, BPW)])


@jax.jit
def _collab_dot(uidx, pidx, table):
    run = functools.partial(
        pl.kernel,
        mesh=plsc.VectorSubcoreMesh(core_axis_name="c", subcore_axis_name="s"),
        compiler_params=pltpu.CompilerParams(needs_layout_passes=False),
        out_type=jax.ShapeDtypeStruct((B,), jnp.float32),
        scratch_types=[
            pltpu.VMEM((NCHUNK, CHUNK), jnp.int32),
            pltpu.VMEM((NCHUNK, CHUNK), jnp.int32),
            pltpu.VMEM((2, CHUNK, 128), jnp.float32),
            pltpu.VMEM((2, CHUNK, 128), jnp.float32),
            pltpu.VMEM((BPW,), jnp.float32),
            pltpu.SemaphoreType.DMA((2,)),
        ],
    )(_sc_body)
    return run(uidx, pidx, table)


def kernel(inputs, users_w, products_w):
    uidx = inputs[:, 0].reshape(B // CHUNK, CHUNK)
    pidx = inputs[:, 1].reshape(B // CHUNK, CHUNK)
    table = _tc_pack(users_w.T, products_w.T)
    out = _collab_dot(uidx, pidx, table)
    return out[:, None]
